# baseline (speedup 1.0000x reference)
import functools

import jax
import jax.numpy as jnp
from jax import lax
from jax.experimental import pallas as pl
from jax.experimental.pallas import tpu as pltpu
from jax.experimental.pallas import tpu_sc as plsc

_VOCAB = 1000
_BATCH = 16384
_SEQ = 50
_DIM = 64
_LUT = 1024

_NC = 2
_NS = 16
_NW = _NC * _NS
_BPW = _BATCH // _NW
_L = 16


def _lut_body(table_ref, w_ref, b_ref, out_ref):
    wd = w_ref[1:2, :] - w_ref[0:1, :]
    d = jax.lax.dot_general(
        table_ref[:, :], wd, (((1,), (1,)), ((), ())),
        preferred_element_type=jnp.float32)
    db = b_ref[0:1, 1:2] - b_ref[0:1, 0:1]
    dfull = jnp.concatenate(
        [d, jnp.zeros((_LUT - _VOCAB, 1), jnp.float32)], axis=0)
    out_ref[:, :] = (dfull + db) * (1.0 / _SEQ)


_lut_call = pl.pallas_call(
    _lut_body,
    out_shape=jax.ShapeDtypeStruct((_LUT, 1), jnp.float32),
)


_HALF = (_BPW // 2) * _SEQ


def _sc_body(x_hbm, d_hbm, out_hbm, x_v, d_v, out_v, sem0, sem1):
    wid = lax.axis_index("s") * _NC + lax.axis_index("c")
    base = wid * (_BPW * _SEQ)
    c0 = pltpu.async_copy(
        x_hbm.at[pl.ds(base, _HALF)], x_v.at[pl.ds(0, _HALF)], sem0)
    c1 = pltpu.async_copy(
        x_hbm.at[pl.ds(base + _HALF, _HALF)], x_v.at[pl.ds(_HALF, _HALF)],
        sem1)
    pltpu.sync_copy(d_hbm, d_v)

    iota = lax.iota(jnp.int32, _L)
    iota_s = iota * _SEQ
    iota_2 = iota * 2

    def group(g):
        idx0 = iota_s + g * (_L * _SEQ)
        acc = jnp.zeros((_L,), jnp.float32)
        for s in range(_SEQ):
            xi = plsc.load_gather(x_v, [idx0 + s])
            acc = acc + plsc.load_gather(d_v, [xi])
        p1 = 1.0 / (1.0 + jnp.exp(-acc))
        o = iota_2 + g * (2 * _L)
        plsc.store_scatter(out_v, [o], 1.0 - p1)
        plsc.store_scatter(out_v, [o + 1], p1)

    def body2(i, carry):
        group(i * 2)
        group(i * 2 + 1)
        return carry

    n2 = _BPW // (2 * _L)
    c0.wait()
    lax.fori_loop(0, n2 // 2, body2, 0)
    c1.wait()
    lax.fori_loop(n2 // 2, n2, body2, 0)
    pltpu.sync_copy(out_v, out_hbm.at[pl.ds(wid * (2 * _BPW), 2 * _BPW)])


_sc_call = functools.partial(
    pl.kernel,
    out_type=jax.ShapeDtypeStruct((2 * _BATCH,), jnp.float32),
    mesh=plsc.VectorSubcoreMesh(core_axis_name="c", subcore_axis_name="s"),
    scratch_types=[
        pltpu.VMEM((_BPW * _SEQ,), jnp.int32),
        pltpu.VMEM((_LUT,), jnp.float32),
        pltpu.VMEM((2 * _BPW,), jnp.float32),
        pltpu.SemaphoreType.DMA,
        pltpu.SemaphoreType.DMA,
    ],
    compiler_params=pltpu.CompilerParams(needs_layout_passes=False),
)(_sc_body)


def kernel(x, table, W, b):
    d = _lut_call(table, W, b.reshape(1, 2))
    out = _sc_call(x.reshape(-1), d.reshape(_LUT))
    return out.reshape(_BATCH, 2)


# speedup vs baseline: 1.1575x; 1.1575x over previous
import functools

import jax
import jax.numpy as jnp
from jax import lax
from jax.experimental import pallas as pl
from jax.experimental.pallas import tpu as pltpu
from jax.experimental.pallas import tpu_sc as plsc

_VOCAB = 1000
_BATCH = 16384
_SEQ = 50
_DIM = 64
_LUT = 1024

_NC = 2
_NS = 16
_NW = _NC * _NS
_BPW = _BATCH // _NW
_L = 16


def _lut_body(table_ref, w_ref, b_ref, out_ref):
    wd = w_ref[1:2, :] - w_ref[0:1, :]
    d = jax.lax.dot_general(
        table_ref[:, :], wd, (((1,), (1,)), ((), ())),
        preferred_element_type=jnp.float32)
    db = b_ref[0:1, 1:2] - b_ref[0:1, 0:1]
    dfull = jnp.concatenate(
        [d, jnp.zeros((_LUT - _VOCAB, 1), jnp.float32)], axis=0)
    out_ref[:, :] = ((dfull + db) * (1.0 / _SEQ)).reshape(8, 128)


_lut_call = pl.pallas_call(
    _lut_body,
    out_shape=jax.ShapeDtypeStruct((8, 128), jnp.float32),
)


_Q = 128
_NBLK = _BPW // _Q
_GPB = _Q // _L


def _sc_body(x_hbm, d_hbm, out_hbm, xq0, xq1, d_v, out_v, sem0, sem1):
    wid = lax.axis_index("s") * _NC + lax.axis_index("c")
    r0 = wid * _BPW
    xq = [xq0, xq1]
    sems = [sem0, sem1]
    cps = [
        pltpu.async_copy(
            x_hbm.at[pl.ds(r0 + k * _Q, _Q), :], xq[k], sems[k])
        for k in range(2)
    ]
    pltpu.sync_copy(d_hbm, d_v)

    iota = lax.iota(jnp.int32, _L)
    zeros_i = jnp.zeros((_L,), jnp.int32)
    ones_i = zeros_i + 1

    for blk in range(_NBLK):
        buf = xq[blk % 2]
        cps[blk % 2].wait()

        def group(g):
            rows_l = iota + g * _L
            acc = jnp.zeros((_L,), jnp.float32)
            for s in range(_SEQ):
                xi = plsc.load_gather(buf, [rows_l, zeros_i + s])
                dv = plsc.load_gather(
                    d_v, [lax.shift_right_logical(xi, 7),
                          lax.bitwise_and(xi, 127)])
                acc = acc + dv
            p1 = 1.0 / (1.0 + jnp.exp(-acc))
            rows = rows_l + blk * _Q
            plsc.store_scatter(out_v, [rows, zeros_i], 1.0 - p1)
            plsc.store_scatter(out_v, [rows, ones_i], p1)

        def body2(i, carry):
            group(i * 2)
            group(i * 2 + 1)
            return carry

        lax.fori_loop(0, _GPB // 2, body2, 0)
        if blk + 2 < _NBLK:
            cps[blk % 2] = pltpu.async_copy(
                x_hbm.at[pl.ds(r0 + (blk + 2) * _Q, _Q), :], buf,
                sems[blk % 2])

    pltpu.sync_copy(out_v, out_hbm.at[pl.ds(r0, _BPW), :])


_sc_call = functools.partial(
    pl.kernel,
    out_type=jax.ShapeDtypeStruct((_BATCH, 2), jnp.float32),
    mesh=plsc.VectorSubcoreMesh(core_axis_name="c", subcore_axis_name="s"),
    scratch_types=[
        pltpu.VMEM((_Q, _SEQ), jnp.int32),
        pltpu.VMEM((_Q, _SEQ), jnp.int32),
        pltpu.VMEM((8, 128), jnp.float32),
        pltpu.VMEM((_BPW, 2), jnp.float32),
        pltpu.SemaphoreType.DMA,
        pltpu.SemaphoreType.DMA,
    ],
    compiler_params=pltpu.CompilerParams(
        needs_layout_passes=False, use_tc_tiling_on_sc=True),
)(_sc_body)


def kernel(x, table, W, b):
    d = _lut_call(table, W, b.reshape(1, 2))
    return _sc_call(x, d)


# speedup vs baseline: 1.1601x; 1.0022x over previous
import functools

import jax
import jax.numpy as jnp
from jax import lax
from jax.experimental import pallas as pl
from jax.experimental.pallas import tpu as pltpu
from jax.experimental.pallas import tpu_sc as plsc

_VOCAB = 1000
_BATCH = 16384
_SEQ = 50
_DIM = 64
_LUT = 1024

_NC = 2
_NS = 16
_NW = _NC * _NS
_BPW = _BATCH // _NW
_L = 16


def _lut_body(table_ref, w_ref, b_ref, out_ref):
    wd = w_ref[1:2, :] - w_ref[0:1, :]
    d = jax.lax.dot_general(
        table_ref[:, :], wd, (((1,), (1,)), ((), ())),
        preferred_element_type=jnp.float32)
    db = b_ref[0:1, 1:2] - b_ref[0:1, 0:1]
    dfull = jnp.concatenate(
        [d, jnp.zeros((_LUT - _VOCAB, 1), jnp.float32)], axis=0)
    out_ref[:, :] = ((dfull + db) * (1.0 / _SEQ)).reshape(8, 128)


_lut_call = pl.pallas_call(
    _lut_body,
    out_shape=jax.ShapeDtypeStruct((8, 128), jnp.float32),
)


_HALF = (_BPW // 2) * _SEQ


def _sc_body(x_hbm, d_hbm, out_hbm, x_v, d_v, out_v, sem0, sem1):
    wid = lax.axis_index("s") * _NC + lax.axis_index("c")
    base = wid * (_BPW * _SEQ)
    c0 = pltpu.async_copy(
        x_hbm.at[pl.ds(base, _HALF)], x_v.at[pl.ds(0, _HALF)], sem0)
    c1 = pltpu.async_copy(
        x_hbm.at[pl.ds(base + _HALF, _HALF)], x_v.at[pl.ds(_HALF, _HALF)],
        sem1)
    pltpu.sync_copy(d_hbm, d_v)

    iota = lax.iota(jnp.int32, _L)
    iota_s = iota * _SEQ
    zeros_i = jnp.zeros((_L,), jnp.int32)
    ones_i = zeros_i + 1

    def group(g):
        idx0 = iota_s + g * (_L * _SEQ)
        acc = jnp.zeros((_L,), jnp.float32)
        for s in range(_SEQ):
            xi = plsc.load_gather(x_v, [idx0 + s])
            dv = plsc.load_gather(
                d_v, [lax.shift_right_logical(xi, 7),
                      lax.bitwise_and(xi, 127)])
            acc = acc + dv
        p1 = 1.0 / (1.0 + jnp.exp(-acc))
        rows = iota + g * _L
        plsc.store_scatter(out_v, [rows, zeros_i], 1.0 - p1)
        plsc.store_scatter(out_v, [rows, ones_i], p1)

    def body2(i, carry):
        group(i * 2)
        group(i * 2 + 1)
        return carry

    n2 = _BPW // (2 * _L)
    c0.wait()
    lax.fori_loop(0, n2 // 2, body2, 0)
    c1.wait()
    lax.fori_loop(n2 // 2, n2, body2, 0)
    pltpu.sync_copy(out_v, out_hbm.at[pl.ds(wid * _BPW, _BPW), :])


_sc_call = functools.partial(
    pl.kernel,
    out_type=jax.ShapeDtypeStruct((_BATCH, 2), jnp.float32),
    mesh=plsc.VectorSubcoreMesh(core_axis_name="c", subcore_axis_name="s"),
    scratch_types=[
        pltpu.VMEM((_BPW * _SEQ,), jnp.int32),
        pltpu.VMEM((8, 128), jnp.float32),
        pltpu.VMEM((_BPW, 2), jnp.float32),
        pltpu.SemaphoreType.DMA,
        pltpu.SemaphoreType.DMA,
    ],
    compiler_params=pltpu.CompilerParams(
        needs_layout_passes=False, use_tc_tiling_on_sc=True),
)(_sc_body)


def kernel(x, table, W, b):
    d = _lut_call(table, W, b.reshape(1, 2))
    return _sc_call(x.reshape(-1), d)
